# baseline (speedup 1.0000x reference)
import jax
import jax.numpy as jnp
from jax import lax
from jax.experimental import pallas as pl
from jax.experimental.pallas import tpu as pltpu
from jax.experimental.pallas import tpu_sc as plsc

NC = 2
NS = 16
L = 16
NW = NC * NS

B = 16384
D = 32
SR = 128
BPW = B // NW
HALF = BPW // 2
HBLOCKS = HALF // L


def _rsqrt_nr(x):
    i = plsc.bitcast(x, jnp.int32)
    i = jnp.int32(0x5F3759DF) - lax.shift_right_logical(i, 1)
    y = plsc.bitcast(i, jnp.float32)
    xh = x * jnp.float32(0.5)
    for _ in range(3):
        y = y * (jnp.float32(1.5) - xh * y * y)
    return y


def _sc_transe(node_hbm, rel_hbm, hsup_hbm, hoff_hbm, tsup_hbm, toff_hbm,
               rsup_hbm, roff_hbm, out_hbm,
               hsv, hov, tsv, tov, rsv, rov, hbuf, tbuf, rbuf,
               outv, s1, s2, s3):
    wid = lax.axis_index("s") * NC + lax.axis_index("c")
    base = wid * BPW

    iota = lax.iota(jnp.int32, L)
    zero = jnp.zeros((L,), jnp.float32)

    for half in range(2):
        cb = base + half * HALF
        pltpu.sync_copy(hsup_hbm.at[pl.ds(cb, HALF)], hsv)
        pltpu.sync_copy(hoff_hbm.at[pl.ds(cb, HALF)], hov)
        pltpu.sync_copy(tsup_hbm.at[pl.ds(cb, HALF)], tsv)
        pltpu.sync_copy(toff_hbm.at[pl.ds(cb, HALF)], tov)
        pltpu.sync_copy(rsup_hbm.at[pl.ds(cb, HALF)], rsv)
        pltpu.sync_copy(roff_hbm.at[pl.ds(cb, HALF)], rov)

        c1 = pltpu.async_copy(node_hbm.at[hsv], hbuf, s1)
        c2 = pltpu.async_copy(node_hbm.at[tsv], tbuf, s2)
        c3 = pltpu.async_copy(rel_hbm.at[rsv], rbuf, s3)
        c1.wait()
        c2.wait()
        c3.wait()

        def block(b, _):
            ridx = b * L + iota
            sl16 = pl.ds(b * L, L)
            hof = hov[sl16]
            tof = tov[sl16]
            rof = rov[sl16]
            hh = tt = rr = hr = ht = rt = zero
            for d in range(D):
                h = plsc.load_gather(hbuf, [ridx, hof + d])
                t = plsc.load_gather(tbuf, [ridx, tof + d])
                r = plsc.load_gather(rbuf, [ridx, rof + d])
                hh = hh + h * h
                tt = tt + t * t
                rr = rr + r * r
                hr = hr + h * r
                ht = ht + h * t
                rt = rt + r * t
            irh = _rsqrt_nr(jnp.maximum(hh, jnp.float32(1e-24)))
            irt = _rsqrt_nr(jnp.maximum(tt, jnp.float32(1e-24)))
            aa = hh * irh * irh
            bb = tt * irt * irt
            cross = hr * irh - ht * (irh * irt) - rt * irt
            dd = aa + bb + rr + (cross + cross)
            s = jnp.maximum(dd, jnp.float32(0.0))
            res = -(s * _rsqrt_nr(jnp.maximum(s, jnp.float32(1e-30))))
            outv[pl.ds(half * HALF + b * L, L)] = res
            return _

        lax.fori_loop(0, HBLOCKS, block, None)

    pltpu.sync_copy(outv, out_hbm.at[pl.ds(base, BPW)])


@jax.jit
def _transe_sc(node_sr, rel_sr, hsup, hoff, tsup, toff, rsup, roff):
    mesh = plsc.VectorSubcoreMesh(
        core_axis_name="c", subcore_axis_name="s",
        num_cores=NC, num_subcores=NS)
    f = pl.kernel(
        _sc_transe,
        out_type=jax.ShapeDtypeStruct((B,), jnp.float32),
        mesh=mesh,
        compiler_params=pltpu.CompilerParams(
            needs_layout_passes=False, use_tc_tiling_on_sc=False),
        scratch_types=[
            pltpu.VMEM((HALF,), jnp.int32),
            pltpu.VMEM((HALF,), jnp.int32),
            pltpu.VMEM((HALF,), jnp.int32),
            pltpu.VMEM((HALF,), jnp.int32),
            pltpu.VMEM((HALF,), jnp.int32),
            pltpu.VMEM((HALF,), jnp.int32),
            pltpu.VMEM((HALF, SR), jnp.float32),
            pltpu.VMEM((HALF, SR), jnp.float32),
            pltpu.VMEM((HALF, SR), jnp.float32),
            pltpu.VMEM((BPW,), jnp.float32),
            pltpu.SemaphoreType.DMA,
            pltpu.SemaphoreType.DMA,
            pltpu.SemaphoreType.DMA,
        ],
    )
    return f(node_sr, rel_sr, hsup, hoff, tsup, toff, rsup, roff)


def kernel(head_index, rel_type, tail_index, node_emb, rel_emb):
    hidx = head_index.astype(jnp.int32)
    ridx = rel_type.astype(jnp.int32)
    tidx = tail_index.astype(jnp.int32)
    node_sr = node_emb.reshape(node_emb.shape[0] // 4, SR)
    rel_sr = rel_emb.reshape(rel_emb.shape[0] // 4, SR)
    return _transe_sc(
        node_sr, rel_sr,
        hidx >> 2, (hidx & 3) * D,
        tidx >> 2, (tidx & 3) * D,
        ridx >> 2, (ridx & 3) * D,
    )
